# Initial kernel scaffold; baseline (speedup 1.0000x reference)
import jax
import jax.numpy as jnp
from jax.experimental import pallas as pl


def kernel(x, positions, embedding_weight):
    raise NotImplementedError("write your pallas kernel here")


# speedup vs baseline: 1.1701x; 1.1701x over previous
import functools

import jax
import jax.numpy as jnp
from jax import lax
from jax.experimental import pallas as pl
from jax.experimental.pallas import tpu as pltpu
from jax.experimental.pallas import tpu_sc as plsc

NC = 2
NS = 16
NW = NC * NS
L = 16
C = 128


def kernel(x, positions, embedding_weight):
    B, S, D = x.shape
    V = embedding_weight.shape[0]
    N = B * S
    xf = x.reshape(N, D)
    pf = positions.reshape(N)
    tabf = embedding_weight.reshape(V * D)
    R = N // NW

    mesh = plsc.VectorSubcoreMesh(core_axis_name="c", subcore_axis_name="s")

    @functools.partial(
        pl.kernel,
        out_type=jax.ShapeDtypeStruct((N, D), jnp.float32),
        mesh=mesh,
        compiler_params=pltpu.CompilerParams(needs_layout_passes=False),
        scratch_types=[
            pltpu.VMEM((V * D,), jnp.float32),
            pltpu.VMEM((C,), jnp.int32),
            pltpu.VMEM((C, D), jnp.float32),
        ],
    )
    def sc_kernel(x_hbm, pos_hbm, tab_hbm, out_hbm, tab_v, idx_v, xbuf):
        cid = lax.axis_index("c")
        sid = lax.axis_index("s")
        wid = sid * NC + cid
        base0 = wid * R
        pltpu.sync_copy(tab_hbm, tab_v)

        @pl.loop(0, R, step=C)
        def _(r):
            base = base0 + r
            pltpu.sync_copy(pos_hbm.at[pl.ds(base, C)], idx_v)
            pltpu.sync_copy(x_hbm.at[pl.ds(base, C)], xbuf)

            @pl.loop(0, C, step=L)
            def _(g):
                rows = idx_v[pl.ds(g, L)] * D
                dst = lax.iota(jnp.int32, L) + g
                for c in range(D):
                    csplat = jnp.full((L,), c, jnp.int32)
                    vals = plsc.load_gather(tab_v, [rows + c])
                    plsc.addupdate_scatter(xbuf, [dst, csplat], vals)

            pltpu.sync_copy(xbuf, out_hbm.at[pl.ds(base, C)])

    out = sc_kernel(xf, pf, tabf)
    return out.reshape(B, S, D)
